# baseline (speedup 1.0000x reference)
import jax
import jax.numpy as jnp
from jax import lax
from jax.experimental import pallas as pl
from jax.experimental.pallas import tpu as pltpu
from jax.experimental.pallas import tpu_sc as plsc

N = 10000
E = 320000
NW = 32
R = 313
NPAD = NW * R
CAP = 11264
CHUNK = 1600
NCHUNK_PAIRS = (E // CHUNK) // 2
GCHUNK = 1024
GSUB = 128
BATCH = 16
ADPAD = 10032
NEG_BIG = -3.0e38

_SC_PARAMS = pltpu.CompilerParams(needs_layout_passes=False)


def _make_tc(din, dout, do_relu):
    blk = 400

    def body(x_ref, w_ref, ws_ref, wd_ref, b_ref, h_ref, as_ref, ad_ref):
        xb = x_ref[...]
        if do_relu:
            xb = jnp.maximum(xb + b_ref[...], 0.0)
        h_ref[...] = jnp.dot(xb, w_ref[...], preferred_element_type=jnp.float32)
        as_ref[...] = jnp.sum(xb * ws_ref[...], axis=1)[None, None, :]
        ad_ref[...] = jnp.sum(xb * wd_ref[...], axis=1)[None, None, :]

    grid = (N // blk,)
    return pl.pallas_call(
        body,
        grid=grid,
        in_specs=[
            pl.BlockSpec((blk, din), lambda i: (i, 0)),
            pl.BlockSpec((din, dout), lambda i: (0, 0)),
            pl.BlockSpec((1, din), lambda i: (0, 0)),
            pl.BlockSpec((1, din), lambda i: (0, 0)),
            pl.BlockSpec((1, din), lambda i: (0, 0)),
        ],
        out_specs=[
            pl.BlockSpec((blk, dout), lambda i: (i, 0)),
            pl.BlockSpec((1, 1, blk), lambda i: (i, 0, 0)),
            pl.BlockSpec((1, 1, blk), lambda i: (i, 0, 0)),
        ],
        out_shape=[
            jax.ShapeDtypeStruct((N, dout), jnp.float32),
            jax.ShapeDtypeStruct((N // blk, 1, blk), jnp.float32),
            jax.ShapeDtypeStruct((N // blk, 1, blk), jnp.float32),
        ],
    )


def _final_bias():
    blk = 400

    def body(z_ref, b_ref, o_ref):
        o_ref[...] = z_ref[...] + b_ref[...]

    return pl.pallas_call(
        body,
        grid=(N // blk,),
        in_specs=[
            pl.BlockSpec((blk, 128), lambda i: (i, 0)),
            pl.BlockSpec((1, 128), lambda i: (0, 0)),
        ],
        out_specs=pl.BlockSpec((blk, 128), lambda i: (i, 0)),
        out_shape=jax.ShapeDtypeStruct((N, 128), jnp.float32),
    )


def _make_sc(D, do_scan, phases=3):
    mesh = plsc.VectorSubcoreMesh(core_axis_name="c", subcore_axis_name="s")
    Dq = D // 4
    out_type = [jax.ShapeDtypeStruct((NPAD * Dq,), jnp.float32)
                for _ in range(4)]
    if do_scan:
        out_type += [
            jax.ShapeDtypeStruct((NW, CAP), jnp.int32),
            jax.ShapeDtypeStruct((NW, CAP), jnp.int32),
            jax.ShapeDtypeStruct((NW, 16), jnp.int32),
        ]
    scratch = [
        pltpu.VMEM((CAP,), jnp.int32),
        pltpu.VMEM((CAP,), jnp.int32),
        pltpu.VMEM((CAP,), jnp.float32),
        pltpu.VMEM((CHUNK,), jnp.int32),
        pltpu.VMEM((CHUNK,), jnp.int32),
        pltpu.VMEM((CHUNK,), jnp.int32),
        pltpu.VMEM((CHUNK,), jnp.int32),
        pltpu.VMEM((GCHUNK,), jnp.float32),
        pltpu.VMEM((336,), jnp.float32),
        pltpu.VMEM((R + 7,), jnp.float32),
        pltpu.VMEM((R * Dq,), jnp.float32),
        pltpu.VMEM((R * Dq,), jnp.float32),
        pltpu.VMEM((R * Dq,), jnp.float32),
        pltpu.VMEM((R * Dq,), jnp.float32),
        pltpu.VMEM((BATCH, D), jnp.float32),
        pltpu.VMEM((BATCH, D), jnp.float32),
        pltpu.VMEM((16,), jnp.int32),
        pltpu.SemaphoreType.DMA,
        pltpu.SemaphoreType.DMA,
        pltpu.SemaphoreType.DMA,
        pltpu.SemaphoreType.DMA,
        pltpu.SemaphoreType.DMA,
        pltpu.SemaphoreType.DMA,
    ]

    def body(*refs):
        if do_scan:
            (src_hbm, dst_hbm, h_hbm, as_hbm, ad_hbm,
             z0_out, z1_out, z2_out, z3_out, sl_out, dl_out, cnt_out,
             src_l, dst_l, alpha_l, eb_s0, eb_d0, eb_s1, eb_d1,
             asg, ad_v, denom_v, oq0, oq1, oq2, oq3, hb0, hb1, cnt16,
             sem0, sem1, sem2, sem3, sem4, sem5) = refs
        else:
            (sl_in, dl_in, cnt_in, h_hbm, as_hbm, ad_hbm,
             z0_out, z1_out, z2_out, z3_out,
             src_l, dst_l, alpha_l, eb_s0, eb_d0, eb_s1, eb_d1,
             asg, ad_v, denom_v, oq0, oq1, oq2, oq3, hb0, hb1, cnt16,
             sem0, sem1, sem2, sem3, sem4, sem5) = refs
        out_q = (oq0, oq1, oq2, oq3)
        z_outs = (z0_out, z1_out, z2_out, z3_out)

        w = lax.axis_index("c") * 16 + lax.axis_index("s")
        base = w * R
        iota16 = lax.iota(jnp.int32, 16)
        zeros16i = jnp.zeros((16,), jnp.int32)
        zeros16f = jnp.zeros((16,), jnp.float32)

        if do_scan:
            base_v = jnp.full((16,), base, jnp.int32)
            hi_v = base_v + R

            def zlists(i, _):
                src_l[pl.ds(i * 16, 16)] = zeros16i
                dst_l[pl.ds(i * 16, 16)] = zeros16i
                return 0

            lax.fori_loop(0, CAP // 16, zlists, 0)

            def start_chunk(t, es, ed, ss, sd):
                off = t * CHUNK
                pltpu.async_copy(src_hbm.at[pl.ds(off, CHUNK)], es, ss)
                pltpu.async_copy(dst_hbm.at[pl.ds(off, CHUNK)], ed, sd)

            def wait_chunk(es, ed, ss, sd):
                pltpu.make_async_copy(
                    src_hbm.at[pl.ds(0, CHUNK)], es, ss).wait()
                pltpu.make_async_copy(
                    dst_hbm.at[pl.ds(0, CHUNK)], ed, sd).wait()

            def scan_chunk(es, ed, cnt_v):
                def step(i, cv):
                    for k in range(4):
                        off = (i * 4 + k) * 16
                        s16 = es[pl.ds(off, 16)]
                        d16 = ed[pl.ds(off, 16)]
                        msk = (d16 >= base_v) & (d16 < hi_v)
                        cs = plsc.cumsum(msk.astype(jnp.int32))
                        pos = cs + cv - 1
                        plsc.store_scatter(src_l, [pos], s16, mask=msk)
                        plsc.store_scatter(dst_l, [pos], d16 - base_v,
                                           mask=msk)
                        cv = cv + plsc.all_reduce_population_count(msk)
                    return cv

                return lax.fori_loop(0, CHUNK // 64, step, cnt_v)

            start_chunk(0, eb_s0, eb_d0, sem0, sem1)

            def p1_pair(tp, cnt_v):
                t = tp * 2
                wait_chunk(eb_s0, eb_d0, sem0, sem1)
                start_chunk(t + 1, eb_s1, eb_d1, sem2, sem3)
                cnt_v = scan_chunk(eb_s0, eb_d0, cnt_v)
                start_chunk(t + 2, eb_s0, eb_d0, sem0, sem1)
                wait_chunk(eb_s1, eb_d1, sem2, sem3)
                cnt_v = scan_chunk(eb_s1, eb_d1, cnt_v)
                return cnt_v

            cnt_v = lax.fori_loop(0, NCHUNK_PAIRS, p1_pair, zeros16i)
            wait_chunk(eb_s0, eb_d0, sem0, sem1)

            cnt16[...] = cnt_v
            pltpu.sync_copy(src_l, sl_out.at[w])
            pltpu.sync_copy(dst_l, dl_out.at[w])
            pltpu.sync_copy(cnt16, cnt_out.at[w])
            cnt = jnp.max(cnt_v)
        else:
            pltpu.sync_copy(sl_in.at[w], src_l)
            pltpu.sync_copy(dl_in.at[w], dst_l)
            pltpu.sync_copy(cnt_in.at[w], cnt16)
            cnt = jnp.max(cnt16[...])

        if phases < 2:
            def zout0(i, _):
                out_v[pl.ds(i * 16, 16)] = zeros16f
                return 0
            lax.fori_loop(0, R * D // 16, zout0, 0)
            pltpu.sync_copy(out_v, z_out.at[pl.ds(base * D, R * D)])
            return

        cnt_v16 = jnp.full((16,), cnt, jnp.int32)
        cntp = ((cnt + 63) // 64) * 64
        nch = (cntp + GCHUNK - 1) // GCHUNK

        base_al = (base // 16) * 16
        off_v = jnp.full((16,), base - base_al, jnp.int32)
        pltpu.sync_copy(ad_hbm.at[pl.ds(base_al, 336)], ad_v)

        def gather_scores(c):
            for kk in range(GCHUNK // GSUB):
                pltpu.async_copy(
                    as_hbm.at[src_l.at[pl.ds(c * GCHUNK + kk * GSUB, GSUB)]],
                    asg.at[pl.ds(kk * GSUB, GSUB)], sem0)
            for kk in range(GCHUNK // GSUB):
                pltpu.make_async_copy(
                    as_hbm.at[src_l.at[pl.ds(0, GSUB)]],
                    asg.at[pl.ds(kk * GSUB, GSUB)], sem0).wait()

        def p2a_chunk(c, mvec):
            gather_scores(c)

            def inner(i, mv):
                g = c * GCHUNK + i * 16
                t = asg[pl.ds(i * 16, 16)] + plsc.load_gather(
                    ad_v, [dst_l[pl.ds(g, 16)] + off_v])
                e = jnp.where(t >= 0, t, 0.2 * t)
                alpha_l[pl.ds(g, 16)] = e
                valid = (jnp.full((16,), g, jnp.int32) + iota16) < cnt_v16
                return jnp.maximum(mv, jnp.where(valid, e, NEG_BIG))

            return lax.fori_loop(0, GCHUNK // 16, inner, mvec)

        mvec = lax.fori_loop(0, nch, p2a_chunk,
                             jnp.full((16,), NEG_BIG, jnp.float32))
        m_v = jnp.full((16,), jnp.max(mvec), jnp.float32)

        def zden(i, _):
            denom_v[pl.ds(i * 16, 16)] = zeros16f
            return 0

        lax.fori_loop(0, (R + 7 + 15) // 16, zden, 0)

        def p2b(i, _):
            g = i * 16
            e16 = alpha_l[pl.ds(g, 16)]
            dl16 = dst_l[pl.ds(g, 16)]
            valid = (jnp.full((16,), g, jnp.int32) + iota16) < cnt_v16
            p = jnp.where(valid, jnp.exp(e16 - m_v), 0.0)
            alpha_l[pl.ds(g, 16)] = p
            plsc.addupdate_scatter(denom_v, [dl16], p, mask=valid)
            return 0

        lax.fori_loop(0, nch * (GCHUNK // 16), p2b, 0)

        def p2c(i, _):
            g = i * 16
            p16 = alpha_l[pl.ds(g, 16)]
            dn = plsc.load_gather(denom_v, [dst_l[pl.ds(g, 16)]])
            alpha_l[pl.ds(g, 16)] = p16 / (dn + 1e-16)
            return 0

        lax.fori_loop(0, nch * (GCHUNK // 16), p2c, 0)

        def zero_and_maybe_write(write):
            for q in range(4):
                oq = out_q[q]

                def zq(i, _):
                    oq[pl.ds(i * 16, 16)] = zeros16f
                    return 0

                lax.fori_loop(0, R * Dq // 16, zq, 0)
                if write:
                    pltpu.sync_copy(
                        oq, z_outs[q].at[pl.ds(base * Dq, R * Dq)])

        if phases < 3:
            zero_and_maybe_write(True)
            return

        zero_and_maybe_write(False)

        def gather_batch(b, hb, sem):
            pltpu.async_copy(
                h_hbm.at[src_l.at[pl.ds(b * BATCH, BATCH)]], hb, sem)

        def wait_batch(b, hb, sem):
            pltpu.make_async_copy(
                h_hbm.at[src_l.at[pl.ds(b * BATCH, BATCH)]], hb, sem).wait()

        def process_batch(b, hb):
            gq = Dq // 16
            for j in range(BATCH):
                eidx = jnp.full((16,), b * BATCH + j, jnp.int32)
                av = plsc.load_gather(alpha_l, [eidx])
                bQv = plsc.load_gather(dst_l, [eidx]) * Dq
                for gp in range(gq):
                    idx = bQv + (gp * 16 + iota16)
                    for q in range(4):
                        rv = hb[j, pl.ds((q * gq + gp) * 16, 16)]
                        plsc.addupdate_scatter(out_q[q], [idx], rv * av)

        npairs = cntp // (2 * BATCH)
        gather_batch(0, hb0, sem4)

        def p3_pair(i, _):
            b = i * 2
            wait_batch(b, hb0, sem4)
            gather_batch(b + 1, hb1, sem5)
            if phases != 4:
                process_batch(b, hb0)
            gather_batch(b + 2, hb0, sem4)
            wait_batch(b + 1, hb1, sem5)
            if phases != 4:
                process_batch(b + 1, hb1)
            return 0

        lax.fori_loop(0, npairs, p3_pair, 0)
        wait_batch(npairs * 2, hb0, sem4)

        for q in range(4):
            pltpu.sync_copy(out_q[q],
                            z_outs[q].at[pl.ds(base * Dq, R * Dq)])

    return pl.kernel(body, out_type=out_type, mesh=mesh,
                     compiler_params=_SC_PARAMS, scratch_types=scratch)


_tc1 = _make_tc(128, 256, do_relu=False)
_tc2 = _make_tc(256, 128, do_relu=True)
_SC_PHASES = 3
_sc_scan = _make_sc(256, do_scan=True, phases=_SC_PHASES)
_sc_agg = _make_sc(128, do_scan=False, phases=_SC_PHASES)
_final = _final_bias()


def kernel(x, edge_index, W1, a_src1, a_dst1, b1, W2, a_src2, a_dst2, b2):
    ws1 = W1 @ a_src1
    wd1 = W1 @ a_dst1
    ws2 = W2 @ a_src2
    wd2 = W2 @ a_dst2
    src = edge_index[0].astype(jnp.int32)
    dst = edge_index[1].astype(jnp.int32)
    srcp = jnp.concatenate([src, jnp.zeros((CHUNK,), jnp.int32)])
    dstp = jnp.concatenate([dst, jnp.zeros((CHUNK,), jnp.int32)])

    zeros128 = jnp.zeros((1, 128), jnp.float32)
    h1, as1, ad1 = _tc1(x, W1, ws1.reshape(1, -1), wd1.reshape(1, -1),
                        zeros128)
    as1f = as1.reshape(-1)
    ad1p = jnp.pad(ad1.reshape(-1), (0, ADPAD - N))

    za, zb, zc, zd, sl, dl, cn = _sc_scan(srcp, dstp, h1, as1f, ad1p)
    z1 = jnp.concatenate(
        [t.reshape(NPAD, 64) for t in (za, zb, zc, zd)], axis=1)

    h2, as2, ad2 = _tc2(z1, W2, ws2.reshape(1, -1), wd2.reshape(1, -1),
                        b1.reshape(1, -1))
    as2f = as2.reshape(-1)
    ad2p = jnp.pad(ad2.reshape(-1), (0, ADPAD - N))

    wa, wb, wc, wd = _sc_agg(sl, dl, cn, h2, as2f, ad2p)
    z2 = jnp.concatenate(
        [t.reshape(NPAD, 32) for t in (wa, wb, wc, wd)], axis=1)

    return _final(z2[:N], b2.reshape(1, -1))


# speedup vs baseline: 1.2816x; 1.2816x over previous
import jax
import jax.numpy as jnp
from jax import lax
from jax.experimental import pallas as pl
from jax.experimental.pallas import tpu as pltpu
from jax.experimental.pallas import tpu_sc as plsc

N = 10000
E = 320000
NW = 32
R = 313
NPAD = NW * R
CAP = 11264
CHUNK = 1600
NCHUNK_PAIRS = (E // CHUNK) // 2
GCHUNK = 1024
GSUB = 128
BATCH = 16
ADPAD = 10032
NEG_BIG = -3.0e38

_SC_PARAMS = pltpu.CompilerParams(needs_layout_passes=False)


def _make_tc(din, dout, do_relu):
    blk = 400

    def body(x_ref, w_ref, ws_ref, wd_ref, b_ref, h_ref, as_ref, ad_ref):
        xb = x_ref[...]
        if do_relu:
            xb = jnp.maximum(xb + b_ref[...], 0.0)
        h_ref[...] = jnp.dot(xb, w_ref[...],
                             preferred_element_type=jnp.float32)
        as_ref[...] = jnp.sum(xb * ws_ref[...], axis=1)[None, None, :]
        ad_ref[...] = jnp.sum(xb * wd_ref[...], axis=1)[None, None, :]

    h_cols = dout
    h_dtype = jnp.float32
    return pl.pallas_call(
        body,
        grid=(N // blk,),
        in_specs=[
            pl.BlockSpec((blk, din), lambda i: (i, 0)),
            pl.BlockSpec((din, dout), lambda i: (0, 0)),
            pl.BlockSpec((1, din), lambda i: (0, 0)),
            pl.BlockSpec((1, din), lambda i: (0, 0)),
            pl.BlockSpec((1, din), lambda i: (0, 0)),
        ],
        out_specs=[
            pl.BlockSpec((blk, h_cols), lambda i: (i, 0)),
            pl.BlockSpec((1, 1, blk), lambda i: (i, 0, 0)),
            pl.BlockSpec((1, 1, blk), lambda i: (i, 0, 0)),
        ],
        out_shape=[
            jax.ShapeDtypeStruct((N, h_cols), h_dtype),
            jax.ShapeDtypeStruct((N // blk, 1, blk), jnp.float32),
            jax.ShapeDtypeStruct((N // blk, 1, blk), jnp.float32),
        ],
    )


def _final_bias():
    blk = 400

    def body(z_ref, b_ref, o_ref):
        o_ref[...] = z_ref[...] + b_ref[...]

    return pl.pallas_call(
        body,
        grid=(N // blk,),
        in_specs=[
            pl.BlockSpec((blk, 128), lambda i: (i, 0)),
            pl.BlockSpec((1, 128), lambda i: (0, 0)),
        ],
        out_specs=pl.BlockSpec((blk, 128), lambda i: (i, 0)),
        out_shape=jax.ShapeDtypeStruct((N, 128), jnp.float32),
    )


def _make_sc(D, do_scan, packed):
    mesh = plsc.VectorSubcoreMesh(core_axis_name="c", subcore_axis_name="s")
    h_cols = D // 2 if packed else D
    h_dtype = jnp.int32 if packed else jnp.float32
    out_type = [jax.ShapeDtypeStruct((NPAD * D,), jnp.float32)]
    if do_scan:
        out_type += [
            jax.ShapeDtypeStruct((NW, CAP), jnp.int32),
            jax.ShapeDtypeStruct((NW, CAP), jnp.int32),
            jax.ShapeDtypeStruct((NW, 16), jnp.int32),
        ]
    scratch = [
        pltpu.VMEM((CAP,), jnp.int32),
        pltpu.VMEM((CAP,), jnp.int32),
        pltpu.VMEM((CAP,), jnp.float32),
        pltpu.VMEM((CHUNK,), jnp.int32),
        pltpu.VMEM((CHUNK,), jnp.int32),
        pltpu.VMEM((CHUNK,), jnp.int32),
        pltpu.VMEM((CHUNK,), jnp.int32),
        pltpu.VMEM((GCHUNK,), jnp.float32),
        pltpu.VMEM((336,), jnp.float32),
        pltpu.VMEM((R + 7,), jnp.float32),
        pltpu.VMEM((R * D,), jnp.float32),
        pltpu.VMEM((BATCH, h_cols), h_dtype),
        pltpu.VMEM((BATCH, h_cols), h_dtype),
        pltpu.VMEM((16,), jnp.int32),
        pltpu.SemaphoreType.DMA,
        pltpu.SemaphoreType.DMA,
        pltpu.SemaphoreType.DMA,
        pltpu.SemaphoreType.DMA,
        pltpu.SemaphoreType.DMA,
        pltpu.SemaphoreType.DMA,
    ]

    def body(*refs):
        if do_scan:
            (src_hbm, dst_hbm, h_hbm, as_hbm, ad_hbm,
             z_out, sl_out, dl_out, cnt_out,
             src_l, dst_l, alpha_l, eb_s0, eb_d0, eb_s1, eb_d1,
             asg, ad_v, denom_v, out_v, hb0, hb1, cnt16,
             sem0, sem1, sem2, sem3, sem4, sem5) = refs
        else:
            (sl_in, dl_in, cnt_in, h_hbm, as_hbm, ad_hbm,
             z_out,
             src_l, dst_l, alpha_l, eb_s0, eb_d0, eb_s1, eb_d1,
             asg, ad_v, denom_v, out_v, hb0, hb1, cnt16,
             sem0, sem1, sem2, sem3, sem4, sem5) = refs

        w = lax.axis_index("c") * 16 + lax.axis_index("s")
        base = w * R
        iota16 = lax.iota(jnp.int32, 16)
        zeros16i = jnp.zeros((16,), jnp.int32)
        zeros16f = jnp.zeros((16,), jnp.float32)

        if do_scan:
            base_v = jnp.full((16,), base, jnp.int32)
            hi_v = base_v + R

            def zlists(i, _):
                src_l[pl.ds(i * 16, 16)] = zeros16i
                dst_l[pl.ds(i * 16, 16)] = zeros16i
                return 0

            lax.fori_loop(0, CAP // 16, zlists, 0)

            def start_chunk(t, es, ed, ss, sd):
                off = t * CHUNK
                pltpu.async_copy(src_hbm.at[pl.ds(off, CHUNK)], es, ss)
                pltpu.async_copy(dst_hbm.at[pl.ds(off, CHUNK)], ed, sd)

            def wait_chunk(es, ed, ss, sd):
                pltpu.make_async_copy(
                    src_hbm.at[pl.ds(0, CHUNK)], es, ss).wait()
                pltpu.make_async_copy(
                    dst_hbm.at[pl.ds(0, CHUNK)], ed, sd).wait()

            def scan_chunk(es, ed, cnt_v):
                def step(i, cv):
                    for k in range(4):
                        off = (i * 4 + k) * 16
                        s16 = es[pl.ds(off, 16)]
                        d16 = ed[pl.ds(off, 16)]
                        msk = (d16 >= base_v) & (d16 < hi_v)
                        cs = plsc.cumsum(msk.astype(jnp.int32))
                        pos = cs + cv - 1
                        plsc.store_scatter(src_l, [pos], s16, mask=msk)
                        plsc.store_scatter(dst_l, [pos], d16 - base_v,
                                           mask=msk)
                        cv = cv + plsc.all_reduce_population_count(msk)
                    return cv

                return lax.fori_loop(0, CHUNK // 64, step, cnt_v)

            start_chunk(0, eb_s0, eb_d0, sem0, sem1)

            def p1_pair(tp, cnt_v):
                t = tp * 2
                wait_chunk(eb_s0, eb_d0, sem0, sem1)
                start_chunk(t + 1, eb_s1, eb_d1, sem2, sem3)
                cnt_v = scan_chunk(eb_s0, eb_d0, cnt_v)
                start_chunk(t + 2, eb_s0, eb_d0, sem0, sem1)
                wait_chunk(eb_s1, eb_d1, sem2, sem3)
                cnt_v = scan_chunk(eb_s1, eb_d1, cnt_v)
                return cnt_v

            cnt_v = lax.fori_loop(0, NCHUNK_PAIRS, p1_pair, zeros16i)
            wait_chunk(eb_s0, eb_d0, sem0, sem1)

            cnt16[...] = cnt_v
            pltpu.sync_copy(src_l, sl_out.at[w])
            pltpu.sync_copy(dst_l, dl_out.at[w])
            pltpu.sync_copy(cnt16, cnt_out.at[w])
            cnt = jnp.max(cnt_v)
        else:
            pltpu.sync_copy(sl_in.at[w], src_l)
            pltpu.sync_copy(dl_in.at[w], dst_l)
            pltpu.sync_copy(cnt_in.at[w], cnt16)
            cnt = jnp.max(cnt16[...])

        cnt_v16 = jnp.full((16,), cnt, jnp.int32)
        cntp = ((cnt + 63) // 64) * 64
        nch = (cntp + GCHUNK - 1) // GCHUNK

        base_al = (base // 16) * 16
        off_v = jnp.full((16,), base - base_al, jnp.int32)
        pltpu.sync_copy(ad_hbm.at[pl.ds(base_al, 336)], ad_v)

        def gather_scores(c):
            for kk in range(GCHUNK // GSUB):
                pltpu.async_copy(
                    as_hbm.at[src_l.at[pl.ds(c * GCHUNK + kk * GSUB, GSUB)]],
                    asg.at[pl.ds(kk * GSUB, GSUB)], sem0)
            for kk in range(GCHUNK // GSUB):
                pltpu.make_async_copy(
                    as_hbm.at[src_l.at[pl.ds(0, GSUB)]],
                    asg.at[pl.ds(kk * GSUB, GSUB)], sem0).wait()

        def p2a_chunk(c, mvec):
            gather_scores(c)

            def inner(i, mv):
                g = c * GCHUNK + i * 16
                t = asg[pl.ds(i * 16, 16)] + plsc.load_gather(
                    ad_v, [dst_l[pl.ds(g, 16)] + off_v])
                e = jnp.where(t >= 0, t, 0.2 * t)
                alpha_l[pl.ds(g, 16)] = e
                valid = (jnp.full((16,), g, jnp.int32) + iota16) < cnt_v16
                return jnp.maximum(mv, jnp.where(valid, e, NEG_BIG))

            return lax.fori_loop(0, GCHUNK // 16, inner, mvec)

        mvec = lax.fori_loop(0, nch, p2a_chunk,
                             jnp.full((16,), NEG_BIG, jnp.float32))
        m_v = jnp.full((16,), jnp.max(mvec), jnp.float32)

        def zden(i, _):
            denom_v[pl.ds(i * 16, 16)] = zeros16f
            return 0

        lax.fori_loop(0, (R + 7 + 15) // 16, zden, 0)

        def p2b(i, _):
            g = i * 16
            e16 = alpha_l[pl.ds(g, 16)]
            dl16 = dst_l[pl.ds(g, 16)]
            valid = (jnp.full((16,), g, jnp.int32) + iota16) < cnt_v16
            p = jnp.where(valid, jnp.exp(e16 - m_v), 0.0)
            alpha_l[pl.ds(g, 16)] = p
            plsc.addupdate_scatter(denom_v, [dl16], p, mask=valid)
            return 0

        lax.fori_loop(0, nch * (GCHUNK // 16), p2b, 0)

        def p2c(i, _):
            g = i * 16
            p16 = alpha_l[pl.ds(g, 16)]
            dn = plsc.load_gather(denom_v, [dst_l[pl.ds(g, 16)]])
            alpha_l[pl.ds(g, 16)] = p16 / (dn + 1e-16)
            return 0

        lax.fori_loop(0, nch * (GCHUNK // 16), p2c, 0)

        def zout(i, _):
            out_v[pl.ds(i * 16, 16)] = zeros16f
            return 0

        lax.fori_loop(0, R * D // 16, zout, 0)

        def gather_batch(b, hb, sem):
            pltpu.async_copy(
                h_hbm.at[src_l.at[pl.ds(b * BATCH, BATCH)]], hb, sem)

        def wait_batch(b, hb, sem):
            pltpu.make_async_copy(
                h_hbm.at[src_l.at[pl.ds(b * BATCH, BATCH)]], hb, sem).wait()

        def process_batch(b, hb):
            def edge(j, _):
                eidx = jnp.full((16,), b * BATCH + j, jnp.int32)
                av = plsc.load_gather(alpha_l, [eidx])
                bv = plsc.load_gather(dst_l, [eidx]) * D
                if packed:
                    mask_hi = jnp.full((16,), -65536, jnp.int32)
                    for g in range(D // 32):
                        rp = hb[j, pl.ds(g * 16, 16)]
                        u = plsc.bitcast(rp << 16, jnp.float32)
                        v = plsc.bitcast(rp & mask_hi, jnp.float32)
                        idx_u = bv + (g * 32 + 2 * iota16)
                        plsc.addupdate_scatter(out_v, [idx_u], u * av)
                        plsc.addupdate_scatter(out_v, [idx_u + 1], v * av)
                else:
                    for g in range(D // 16):
                        rv = hb[j, pl.ds(g * 16, 16)]
                        plsc.addupdate_scatter(
                            out_v, [bv + (g * 16 + iota16)], rv * av)
                return 0

            lax.fori_loop(0, BATCH, edge, 0)

        npairs = cntp // (2 * BATCH)
        gather_batch(0, hb0, sem4)

        def p3_pair(i, _):
            b = i * 2
            wait_batch(b, hb0, sem4)
            gather_batch(b + 1, hb1, sem5)
            process_batch(b, hb0)
            gather_batch(b + 2, hb0, sem4)
            wait_batch(b + 1, hb1, sem5)
            process_batch(b + 1, hb1)
            return 0

        lax.fori_loop(0, npairs, p3_pair, 0)
        wait_batch(npairs * 2, hb0, sem4)

        pltpu.sync_copy(out_v, z_out.at[pl.ds(base * D, R * D)])

    return pl.kernel(body, out_type=out_type, mesh=mesh,
                     compiler_params=_SC_PARAMS, scratch_types=scratch)


_tc1 = _make_tc(128, 256, do_relu=False)
_tc2 = _make_tc(256, 128, do_relu=True)
_sc_scan = _make_sc(256, do_scan=True, packed=True)
_sc_agg = _make_sc(128, do_scan=False, packed=False)
_final = _final_bias()


def kernel(x, edge_index, W1, a_src1, a_dst1, b1, W2, a_src2, a_dst2, b2):
    ws1 = W1 @ a_src1
    wd1 = W1 @ a_dst1
    ws2 = W2 @ a_src2
    wd2 = W2 @ a_dst2
    src = edge_index[0].astype(jnp.int32)
    dst = edge_index[1].astype(jnp.int32)
    srcp = jnp.concatenate([src, jnp.zeros((CHUNK,), jnp.int32)])
    dstp = jnp.concatenate([dst, jnp.zeros((CHUNK,), jnp.int32)])

    zeros128 = jnp.zeros((1, 128), jnp.float32)
    h1, as1, ad1 = _tc1(x, W1, ws1.reshape(1, -1), wd1.reshape(1, -1),
                        zeros128)
    as1f = as1.reshape(-1)
    ad1p = jnp.pad(ad1.reshape(-1), (0, ADPAD - N))

    bits = lax.bitcast_convert_type(h1, jnp.uint32).reshape(N, 128, 2)
    h1p = lax.bitcast_convert_type(
        (bits[:, :, 0] >> 16) | (bits[:, :, 1] & jnp.uint32(0xFFFF0000)),
        jnp.int32)

    z1f, sl, dl, cn = _sc_scan(srcp, dstp, h1p, as1f, ad1p)
    z1 = z1f.reshape(NPAD, 256)

    h2, as2, ad2 = _tc2(z1, W2, ws2.reshape(1, -1), wd2.reshape(1, -1),
                        b1.reshape(1, -1))
    as2f = as2.reshape(-1)
    ad2p = jnp.pad(ad2.reshape(-1), (0, ADPAD - N))

    (z2f,) = _sc_agg(sl, dl, cn, h2, as2f, ad2p)
    z2 = z2f.reshape(NPAD, 128)

    return _final(z2[:N], b2.reshape(1, -1))
